# Initial kernel scaffold; baseline (speedup 1.0000x reference)
import jax
import jax.numpy as jnp
from jax.experimental import pallas as pl


def kernel(x, edge_index, W1, b1, Wa, ba, Wo, bo):
    raise NotImplementedError("write your pallas kernel here")


# speedup vs baseline: 15.5026x; 15.5026x over previous
import functools

import jax
import jax.numpy as jnp
from jax import lax
from jax.experimental import pallas as pl
from jax.experimental.pallas import tpu as pltpu
from jax.experimental.pallas import tpu_sc as plsc

N_NODES = 10000
NPAD = 10240
SLICE = NPAD // 16
DEAD = 10016
D_IN = 128
NT = 32
LANES = 16
BN = 2000

def _mesh():
    return plsc.VectorSubcoreMesh(core_axis_name="c", subcore_axis_name="s")


@functools.lru_cache(maxsize=None)
def _make_deg(ch):
    @functools.partial(
        pl.kernel,
        mesh=_mesh(),
        compiler_params=pltpu.CompilerParams(
            needs_layout_passes=False, use_tc_tiling_on_sc=False),
        out_type=(jax.ShapeDtypeStruct((NT, NPAD), jnp.float32),
                  jax.ShapeDtypeStruct((2, NPAD), jnp.float32)),
        scratch_types=[
            pltpu.VMEM((ch, 128), jnp.int32),
            pltpu.VMEM((NPAD,), jnp.float32),
            pltpu.VMEM((SLICE,), jnp.float32),
            pltpu.VMEM((SLICE,), jnp.float32),
        ],
    )
    def deg_kernel(pk_hbm, stage_hbm, out_hbm, pk_v, deg_v, acc_v, tmp_v):
        c = lax.axis_index("c")
        s = lax.axis_index("s")
        wid = c * 16 + s
        pltpu.sync_copy(pk_hbm.at[wid], pk_v)
        zero = jnp.zeros((LANES,), jnp.float32)
        one = jnp.ones((LANES,), jnp.float32)

        def zb(i, carry):
            deg_v[pl.ds(i * LANES, LANES)] = zero
            return carry

        lax.fori_loop(0, NPAD // LANES, zb, 0)

        def hb(i, carry):
            p = pk_v[i // 8, pl.ds((i % 8) * LANES, LANES)]
            idx = lax.bitwise_and(p, 16383)
            plsc.addupdate_scatter(deg_v, [idx], one)
            return carry

        lax.fori_loop(0, ch * 8, hb, 0)

        pltpu.sync_copy(deg_v, stage_hbm.at[wid])
        plsc.subcore_barrier()

        def za(i, carry):
            acc_v[pl.ds(i * LANES, LANES)] = zero
            return carry

        lax.fori_loop(0, SLICE // LANES, za, 0)

        def rb(t, carry):
            pltpu.sync_copy(stage_hbm.at[c * 16 + t, pl.ds(s * SLICE, SLICE)],
                            tmp_v)

            def ab(i, c2):
                sl = pl.ds(i * LANES, LANES)
                acc_v[sl] = acc_v[sl] + tmp_v[sl]
                return c2

            return lax.fori_loop(0, SLICE // LANES, ab, carry)

        lax.fori_loop(0, 16, rb, 0)
        pltpu.sync_copy(acc_v, out_hbm.at[c, pl.ds(s * SLICE, SLICE)])

    return deg_kernel


@functools.lru_cache(maxsize=None)
def _make_agg(ch):
    @functools.partial(
        pl.kernel,
        mesh=_mesh(),
        compiler_params=pltpu.CompilerParams(
            needs_layout_passes=False, use_tc_tiling_on_sc=False),
        out_type=jax.ShapeDtypeStruct((2, NPAD, D_IN), jnp.float32),
        scratch_types=[
            pltpu.VMEM((ch, 128), jnp.int32),
            pltpu.VMEM((ch, 128), jnp.int32),
            pltpu.VMEM((128, D_IN), jnp.float32),
            pltpu.VMEM_SHARED((NPAD, D_IN), jnp.float32),
            pltpu.SemaphoreType.DMA,
        ],
    )
    def agg_kernel(pk_hbm, y_hbm, out_hbm, col_v, row_v, gbuf, acc_sh, sem):
        c = lax.axis_index("c")
        s = lax.axis_index("s")
        wid = c * 16 + s
        pltpu.sync_copy(pk_hbm.at[wid], col_v)

        zero = jnp.zeros((LANES,), jnp.float32)

        def ub(i, carry):
            r = i // 8
            sl = pl.ds((i % 8) * LANES, LANES)
            p = col_v[r, sl]
            row_v[r, sl] = lax.shift_right_logical(p, 14)
            col_v[r, sl] = lax.bitwise_and(p, 16383)
            return carry

        lax.fori_loop(0, ch * 8, ub, 0)

        def zb(i, carry):
            gbuf[i // 8, pl.ds((i % 8) * LANES, LANES)] = zero
            return carry

        lax.fori_loop(0, 128 * (D_IN // LANES), zb, 0)
        for k in range(SLICE // 128):
            pltpu.sync_copy(gbuf, acc_sh.at[pl.ds(s * SLICE + k * 128, 128)])
        plsc.subcore_barrier()

        def eb(g, carry):
            pltpu.async_copy(y_hbm.at[row_v.at[g]], gbuf, sem).wait()
            pltpu.sync_copy(gbuf, acc_sh.at[col_v.at[g]], add=True)
            return carry

        lax.fori_loop(0, ch, eb, 0)

        plsc.subcore_barrier()
        pltpu.sync_copy(acc_sh.at[pl.ds(s * SLICE, SLICE)],
                        out_hbm.at[c, pl.ds(s * SLICE, SLICE)])

    return agg_kernel


def _prep_body(pr_ref, x_ref, dis_ref, y_ref):
    deg = pr_ref[0] + pr_ref[1]
    dis = jnp.where(deg > 0.0, lax.rsqrt(deg), 0.0)
    dis_ref[...] = dis
    y_ref[...] = x_ref[...] * dis[:N_NODES]


def _final_body(q_ref, dis_ref, w1t_ref, b1_ref, wat_ref, ba_ref,
                wot_ref, bo_ref, out_ref, acc_ref, m_ref, z_ref):
    i = pl.program_id(0)

    @pl.when(i == 0)
    def _init():
        m_ref[0] = -jnp.inf
        z_ref[0] = 0.0
        acc_ref[...] = jnp.zeros_like(acc_ref)

    agg = (q_ref[0] + q_ref[1]) * dis_ref[...]
    h = jnp.dot(agg, w1t_ref[...], preferred_element_type=jnp.float32)
    h = jnp.maximum(h + b1_ref[...], 0.0)
    sv = jnp.dot(h, wat_ref[...], preferred_element_type=jnp.float32)
    sv = sv + ba_ref[...]
    bm = jnp.max(sv)
    m_old = m_ref[0]
    m_new = jnp.maximum(m_old, bm)
    alpha = jnp.exp(m_old - m_new)
    w = jnp.exp(sv - m_new)
    z_ref[0] = z_ref[0] * alpha + jnp.sum(w)
    acc_ref[...] = acc_ref[...] * alpha + jnp.sum(w * h, axis=0, keepdims=True)
    m_ref[0] = m_new

    @pl.when(i == pl.num_programs(0) - 1)
    def _fin():
        pooled = acc_ref[...] / z_ref[0]
        out_ref[...] = jnp.dot(pooled, wot_ref[...],
                               preferred_element_type=jnp.float32) + bo_ref[...]


def kernel(x, edge_index, W1, b1, Wa, ba, Wo, bo):
    n, d_in = x.shape
    d_hid = W1.shape[0]
    d_out = Wo.shape[0]
    e = edge_index.shape[1]

    ei = edge_index.astype(jnp.int32)
    ar = jnp.arange(n, dtype=jnp.int32)
    row = jnp.concatenate([ei[0], ar, ar])
    col = jnp.concatenate([ei[1], ar, ar])
    etot = e + 2 * n
    ch = -(-etot // (NT * 128))
    pad = NT * 128 * ch - etot
    packed = row * 16384 + col
    packed = jnp.concatenate([packed, jnp.full((pad,), DEAD, jnp.int32)])
    pk3 = packed.reshape(NT, ch, 128)

    _, deg_partial = _make_deg(ch)(pk3)
    pr2 = deg_partial.reshape(2, NPAD, 1)
    dis_col, y = pl.pallas_call(
        _prep_body,
        out_shape=(jax.ShapeDtypeStruct((NPAD, 1), jnp.float32),
                   jax.ShapeDtypeStruct((n, d_in), jnp.float32)),
    )(pr2, x)
    q = _make_agg(ch)(pk3, y)

    out = pl.pallas_call(
        _final_body,
        grid=(n // BN,),
        in_specs=[
            pl.BlockSpec((2, BN, d_in), lambda i: (0, i, 0)),
            pl.BlockSpec((BN, 1), lambda i: (i, 0)),
            pl.BlockSpec((d_in, d_hid), lambda i: (0, 0)),
            pl.BlockSpec((1, d_hid), lambda i: (0, 0)),
            pl.BlockSpec((d_hid, 1), lambda i: (0, 0)),
            pl.BlockSpec((1, 1), lambda i: (0, 0)),
            pl.BlockSpec((d_hid, d_out), lambda i: (0, 0)),
            pl.BlockSpec((1, d_out), lambda i: (0, 0)),
        ],
        out_specs=pl.BlockSpec((1, d_out), lambda i: (0, 0)),
        out_shape=jax.ShapeDtypeStruct((1, d_out), jnp.float32),
        scratch_shapes=[
            pltpu.VMEM((1, d_hid), jnp.float32),
            pltpu.SMEM((1,), jnp.float32),
            pltpu.SMEM((1,), jnp.float32),
        ],
    )(q, dis_col, W1.T, b1.reshape(1, d_hid), Wa.T, ba.reshape(1, 1),
      Wo.T, bo.reshape(1, d_out))
    return out


# speedup vs baseline: 16.7716x; 1.0819x over previous
import functools

import jax
import jax.numpy as jnp
from jax import lax
from jax.experimental import pallas as pl
from jax.experimental.pallas import tpu as pltpu
from jax.experimental.pallas import tpu_sc as plsc

N_NODES = 10000
NPAD = 10240
SLICE = NPAD // 16
DEAD = 10016
D_IN = 128
NT = 32
LANES = 16
CK = 96
BN = 2000

def _mesh():
    return plsc.VectorSubcoreMesh(core_axis_name="c", subcore_axis_name="s")


@functools.lru_cache(maxsize=None)
def _make_deg(ch):
    @functools.partial(
        pl.kernel,
        mesh=_mesh(),
        compiler_params=pltpu.CompilerParams(
            needs_layout_passes=False, use_tc_tiling_on_sc=False),
        out_type=(jax.ShapeDtypeStruct((NT, NPAD), jnp.float32),
                  jax.ShapeDtypeStruct((2, NPAD), jnp.float32)),
        scratch_types=[
            pltpu.VMEM((ch, CK), jnp.int32),
            pltpu.VMEM((NPAD,), jnp.float32),
            pltpu.VMEM((SLICE,), jnp.float32),
            pltpu.VMEM((SLICE,), jnp.float32),
        ],
    )
    def deg_kernel(pk_hbm, stage_hbm, out_hbm, pk_v, deg_v, acc_v, tmp_v):
        c = lax.axis_index("c")
        s = lax.axis_index("s")
        wid = c * 16 + s
        pltpu.sync_copy(pk_hbm.at[wid], pk_v)
        zero = jnp.zeros((LANES,), jnp.float32)
        one = jnp.ones((LANES,), jnp.float32)

        def zb(i, carry):
            deg_v[pl.ds(i * LANES, LANES)] = zero
            return carry

        lax.fori_loop(0, NPAD // LANES, zb, 0)

        def hb(i, carry):
            p = pk_v[i // (CK // LANES), pl.ds((i % (CK // LANES)) * LANES, LANES)]
            idx = lax.bitwise_and(p, 16383)
            plsc.addupdate_scatter(deg_v, [idx], one)
            return carry

        lax.fori_loop(0, ch * (CK // LANES), hb, 0)

        pltpu.sync_copy(deg_v, stage_hbm.at[wid])
        plsc.subcore_barrier()

        def za(i, carry):
            acc_v[pl.ds(i * LANES, LANES)] = zero
            return carry

        lax.fori_loop(0, SLICE // LANES, za, 0)

        def rb(t, carry):
            pltpu.sync_copy(stage_hbm.at[c * 16 + t, pl.ds(s * SLICE, SLICE)],
                            tmp_v)

            def ab(i, c2):
                sl = pl.ds(i * LANES, LANES)
                acc_v[sl] = acc_v[sl] + tmp_v[sl]
                return c2

            return lax.fori_loop(0, SLICE // LANES, ab, carry)

        lax.fori_loop(0, 16, rb, 0)
        pltpu.sync_copy(acc_v, out_hbm.at[c, pl.ds(s * SLICE, SLICE)])

    return deg_kernel


@functools.lru_cache(maxsize=None)
def _make_agg(ch):
    @functools.partial(
        pl.kernel,
        mesh=_mesh(),
        compiler_params=pltpu.CompilerParams(
            needs_layout_passes=False, use_tc_tiling_on_sc=False),
        out_type=jax.ShapeDtypeStruct((2, NPAD, D_IN), jnp.float32),
        scratch_types=[
            pltpu.VMEM((ch, CK), jnp.int32),
            pltpu.VMEM((ch, CK), jnp.int32),
            pltpu.VMEM((CK, D_IN), jnp.float32),
            pltpu.VMEM((CK, D_IN), jnp.float32),
            pltpu.VMEM_SHARED((NPAD, D_IN), jnp.float32),
            pltpu.SemaphoreType.DMA,
            pltpu.SemaphoreType.DMA,
            pltpu.SemaphoreType.DMA,
            pltpu.SemaphoreType.DMA,
        ],
    )
    def agg_kernel(pk_hbm, y_hbm, out_hbm, col_v, row_v, gbuf0, gbuf1,
                   acc_sh, gs0, gs1, ss0, ss1):
        c = lax.axis_index("c")
        s = lax.axis_index("s")
        wid = c * 16 + s
        pltpu.sync_copy(pk_hbm.at[wid], col_v)

        zero = jnp.zeros((LANES,), jnp.float32)
        nsub = CK // LANES

        def ub(i, carry):
            r = i // nsub
            sl = pl.ds((i % nsub) * LANES, LANES)
            p = col_v[r, sl]
            row_v[r, sl] = lax.shift_right_logical(p, 14)
            col_v[r, sl] = lax.bitwise_and(p, 16383)
            return carry

        lax.fori_loop(0, ch * nsub, ub, 0)

        def zb(i, carry):
            gbuf0[i // (D_IN // LANES), pl.ds((i % (D_IN // LANES)) * LANES, LANES)] = zero
            return carry

        lax.fori_loop(0, CK * (D_IN // LANES), zb, 0)
        for k in range(SLICE // CK):
            pltpu.sync_copy(gbuf0, acc_sh.at[pl.ds(s * SLICE + k * CK, CK)])
        rem = SLICE - (SLICE // CK) * CK
        if rem:
            pltpu.sync_copy(
                gbuf0.at[pl.ds(0, rem)],
                acc_sh.at[pl.ds(s * SLICE + (SLICE // CK) * CK, rem)])

        pltpu.async_copy(y_hbm.at[row_v.at[0]], gbuf0, gs0)
        pltpu.async_copy(y_hbm.at[row_v.at[1]], gbuf1, gs1)
        plsc.subcore_barrier()

        def eb(i, carry):
            g0 = i * 2
            g1 = g0 + 1
            pltpu.make_async_copy(y_hbm.at[row_v.at[g0]], gbuf0, gs0).wait()
            pltpu.async_copy(gbuf0, acc_sh.at[col_v.at[g0]], ss0, add=True)
            pltpu.make_async_copy(y_hbm.at[row_v.at[g1]], gbuf1, gs1).wait()
            pltpu.async_copy(gbuf1, acc_sh.at[col_v.at[g1]], ss1, add=True)
            pltpu.make_async_copy(gbuf0, acc_sh.at[col_v.at[g0]], ss0).wait()

            @pl.when(g0 + 2 < ch)
            def _():
                pltpu.async_copy(y_hbm.at[row_v.at[g0 + 2]], gbuf0, gs0)

            pltpu.make_async_copy(gbuf1, acc_sh.at[col_v.at[g1]], ss1).wait()

            @pl.when(g1 + 2 < ch)
            def _():
                pltpu.async_copy(y_hbm.at[row_v.at[g1 + 2]], gbuf1, gs1)

            return carry

        lax.fori_loop(0, ch // 2, eb, 0)

        plsc.subcore_barrier()
        pltpu.sync_copy(acc_sh.at[pl.ds(s * SLICE, SLICE)],
                        out_hbm.at[c, pl.ds(s * SLICE, SLICE)])

    return agg_kernel


def _prep_body(pr_ref, x_ref, dis_ref, y_ref):
    deg = pr_ref[0] + pr_ref[1]
    dis = jnp.where(deg > 0.0, lax.rsqrt(deg), 0.0)
    dis_ref[...] = dis
    y_ref[...] = x_ref[...] * dis[:N_NODES]


def _final_body(q_ref, dis_ref, w1t_ref, b1_ref, wat_ref, ba_ref,
                wot_ref, bo_ref, out_ref, acc_ref, m_ref, z_ref):
    i = pl.program_id(0)

    @pl.when(i == 0)
    def _init():
        m_ref[0] = -jnp.inf
        z_ref[0] = 0.0
        acc_ref[...] = jnp.zeros_like(acc_ref)

    agg = (q_ref[0] + q_ref[1]) * dis_ref[...]
    h = jnp.dot(agg, w1t_ref[...], preferred_element_type=jnp.float32)
    h = jnp.maximum(h + b1_ref[...], 0.0)
    sv = jnp.dot(h, wat_ref[...], preferred_element_type=jnp.float32)
    sv = sv + ba_ref[...]
    bm = jnp.max(sv)
    m_old = m_ref[0]
    m_new = jnp.maximum(m_old, bm)
    alpha = jnp.exp(m_old - m_new)
    w = jnp.exp(sv - m_new)
    z_ref[0] = z_ref[0] * alpha + jnp.sum(w)
    acc_ref[...] = acc_ref[...] * alpha + jnp.sum(w * h, axis=0, keepdims=True)
    m_ref[0] = m_new

    @pl.when(i == pl.num_programs(0) - 1)
    def _fin():
        pooled = acc_ref[...] / z_ref[0]
        out_ref[...] = jnp.dot(pooled, wot_ref[...],
                               preferred_element_type=jnp.float32) + bo_ref[...]


def kernel(x, edge_index, W1, b1, Wa, ba, Wo, bo):
    n, d_in = x.shape
    d_hid = W1.shape[0]
    d_out = Wo.shape[0]
    e = edge_index.shape[1]

    ei = edge_index.astype(jnp.int32)
    ar = jnp.arange(n, dtype=jnp.int32)
    row = jnp.concatenate([ei[0], ar, ar])
    col = jnp.concatenate([ei[1], ar, ar])
    etot = e + 2 * n
    ch = -(-etot // (NT * CK))
    ch += ch % 2
    pad = NT * CK * ch - etot
    packed = row * 16384 + col
    packed = jnp.concatenate([packed, jnp.full((pad,), DEAD, jnp.int32)])
    pk3 = packed.reshape(NT, ch, CK)

    _, deg_partial = _make_deg(ch)(pk3)
    pr2 = deg_partial.reshape(2, NPAD, 1)
    dis_col, y = pl.pallas_call(
        _prep_body,
        out_shape=(jax.ShapeDtypeStruct((NPAD, 1), jnp.float32),
                   jax.ShapeDtypeStruct((n, d_in), jnp.float32)),
    )(pr2, x)
    q = _make_agg(ch)(pk3, y)

    out = pl.pallas_call(
        _final_body,
        grid=(n // BN,),
        in_specs=[
            pl.BlockSpec((2, BN, d_in), lambda i: (0, i, 0)),
            pl.BlockSpec((BN, 1), lambda i: (i, 0)),
            pl.BlockSpec((d_in, d_hid), lambda i: (0, 0)),
            pl.BlockSpec((1, d_hid), lambda i: (0, 0)),
            pl.BlockSpec((d_hid, 1), lambda i: (0, 0)),
            pl.BlockSpec((1, 1), lambda i: (0, 0)),
            pl.BlockSpec((d_hid, d_out), lambda i: (0, 0)),
            pl.BlockSpec((1, d_out), lambda i: (0, 0)),
        ],
        out_specs=pl.BlockSpec((1, d_out), lambda i: (0, 0)),
        out_shape=jax.ShapeDtypeStruct((1, d_out), jnp.float32),
        scratch_shapes=[
            pltpu.VMEM((1, d_hid), jnp.float32),
            pltpu.SMEM((1,), jnp.float32),
            pltpu.SMEM((1,), jnp.float32),
        ],
    )(q, dis_col, W1.T, b1.reshape(1, d_hid), Wa.T, ba.reshape(1, 1),
      Wo.T, bo.reshape(1, d_out))
    return out


# speedup vs baseline: 21.8398x; 1.3022x over previous
import functools

import jax
import jax.numpy as jnp
from jax import lax
from jax.experimental import pallas as pl
from jax.experimental.pallas import tpu as pltpu
from jax.experimental.pallas import tpu_sc as plsc

N_NODES = 10000
NPAD = 10240
SLICE = NPAD // 16
DEAD = 10016
D_IN = 128
NT = 32
LANES = 16
CK = 96
CORE0_FRAC = 0.30
BN = 2000

def _mesh():
    return plsc.VectorSubcoreMesh(core_axis_name="c", subcore_axis_name="s")


@functools.lru_cache(maxsize=None)
def _make_deg(ch):
    @functools.partial(
        pl.kernel,
        mesh=_mesh(),
        compiler_params=pltpu.CompilerParams(
            needs_layout_passes=False, use_tc_tiling_on_sc=False),
        out_type=(jax.ShapeDtypeStruct((NT, NPAD), jnp.float32),
                  jax.ShapeDtypeStruct((2, NPAD), jnp.float32)),
        scratch_types=[
            pltpu.VMEM((ch, CK), jnp.int32),
            pltpu.VMEM((NPAD,), jnp.float32),
            pltpu.VMEM((SLICE,), jnp.float32),
            pltpu.VMEM((SLICE,), jnp.float32),
        ],
    )
    def deg_kernel(pk_hbm, stage_hbm, out_hbm, pk_v, deg_v, acc_v, tmp_v):
        c = lax.axis_index("c")
        s = lax.axis_index("s")
        wid = c * 16 + s
        pltpu.sync_copy(pk_hbm.at[wid], pk_v)
        zero = jnp.zeros((LANES,), jnp.float32)
        one = jnp.ones((LANES,), jnp.float32)

        def zb(i, carry):
            deg_v[pl.ds(i * LANES, LANES)] = zero
            return carry

        lax.fori_loop(0, NPAD // LANES, zb, 0)

        def hb(i, carry):
            p = pk_v[i // (CK // LANES), pl.ds((i % (CK // LANES)) * LANES, LANES)]
            idx = lax.bitwise_and(p, 16383)
            plsc.addupdate_scatter(deg_v, [idx], one)
            return carry

        lax.fori_loop(0, ch * (CK // LANES), hb, 0)

        pltpu.sync_copy(deg_v, stage_hbm.at[wid])
        plsc.subcore_barrier()

        def za(i, carry):
            acc_v[pl.ds(i * LANES, LANES)] = zero
            return carry

        lax.fori_loop(0, SLICE // LANES, za, 0)

        def rb(t, carry):
            pltpu.sync_copy(stage_hbm.at[c * 16 + t, pl.ds(s * SLICE, SLICE)],
                            tmp_v)

            def ab(i, c2):
                sl = pl.ds(i * LANES, LANES)
                acc_v[sl] = acc_v[sl] + tmp_v[sl]
                return c2

            return lax.fori_loop(0, SLICE // LANES, ab, carry)

        lax.fori_loop(0, 16, rb, 0)
        pltpu.sync_copy(acc_v, out_hbm.at[c, pl.ds(s * SLICE, SLICE)])

    return deg_kernel


@functools.lru_cache(maxsize=None)
def _make_agg(ch0, ch1):
    chmax = max(ch0, ch1)

    @functools.partial(
        pl.kernel,
        mesh=_mesh(),
        compiler_params=pltpu.CompilerParams(
            needs_layout_passes=False, use_tc_tiling_on_sc=False),
        out_type=jax.ShapeDtypeStruct((2, NPAD, D_IN), jnp.float32),
        scratch_types=[
            pltpu.VMEM((chmax, CK), jnp.int32),
            pltpu.VMEM((4, CK), jnp.int32),
            pltpu.VMEM((4, CK), jnp.int32),
            pltpu.VMEM((CK, D_IN), jnp.float32),
            pltpu.VMEM((CK, D_IN), jnp.float32),
            pltpu.VMEM_SHARED((NPAD, D_IN), jnp.float32),
            pltpu.SemaphoreType.DMA,
            pltpu.SemaphoreType.DMA,
            pltpu.SemaphoreType.DMA,
            pltpu.SemaphoreType.DMA,
        ],
    )
    def agg_kernel(pk_hbm, y_hbm, out_hbm, pk_v, col_r, row_r, gbuf0, gbuf1,
                   acc_sh, gs0, gs1, ss0, ss1):
        c = lax.axis_index("c")
        s = lax.axis_index("s")
        wid = c * 16 + s
        nch = jnp.where(c == 0, ch0, ch1)
        pltpu.sync_copy(pk_hbm.at[wid], pk_v)

        zero = jnp.zeros((LANES,), jnp.float32)
        nsub = CK // LANES

        def unpack(g):
            slot = lax.rem(g, 4)

            def uu(j, carry):
                sl = pl.ds(j * LANES, LANES)
                p = pk_v[g, sl]
                row_r[slot, sl] = lax.shift_right_logical(p, 14)
                col_r[slot, sl] = lax.bitwise_and(p, 16383)
                return carry

            lax.fori_loop(0, nsub, uu, 0)

        def zb(i, carry):
            gbuf0[i // (D_IN // LANES),
                  pl.ds((i % (D_IN // LANES)) * LANES, LANES)] = zero
            return carry

        lax.fori_loop(0, CK * (D_IN // LANES), zb, 0)
        for k in range(SLICE // CK):
            pltpu.sync_copy(gbuf0, acc_sh.at[pl.ds(s * SLICE + k * CK, CK)])
        rem = SLICE - (SLICE // CK) * CK
        if rem:
            pltpu.sync_copy(
                gbuf0.at[pl.ds(0, rem)],
                acc_sh.at[pl.ds(s * SLICE + (SLICE // CK) * CK, rem)])

        unpack(jnp.int32(0))
        unpack(jnp.int32(1))
        pltpu.async_copy(y_hbm.at[row_r.at[0]], gbuf0, gs0)
        pltpu.async_copy(y_hbm.at[row_r.at[1]], gbuf1, gs1)
        plsc.subcore_barrier()

        def eb(i, carry):
            g0 = i * 2
            g1 = g0 + 1
            s0 = lax.rem(g0, 4)
            s1 = lax.rem(g1, 4)
            pltpu.make_async_copy(y_hbm.at[row_r.at[s0]], gbuf0, gs0).wait()
            pltpu.async_copy(gbuf0, acc_sh.at[col_r.at[s0]], ss0, add=True)
            pltpu.make_async_copy(y_hbm.at[row_r.at[s1]], gbuf1, gs1).wait()
            pltpu.async_copy(gbuf1, acc_sh.at[col_r.at[s1]], ss1, add=True)
            pltpu.make_async_copy(gbuf0, acc_sh.at[col_r.at[s0]], ss0).wait()

            @pl.when(g0 + 2 < nch)
            def _():
                unpack(g0 + 2)
                pltpu.async_copy(y_hbm.at[row_r.at[lax.rem(g0 + 2, 4)]],
                                 gbuf0, gs0)

            pltpu.make_async_copy(gbuf1, acc_sh.at[col_r.at[s1]], ss1).wait()

            @pl.when(g1 + 2 < nch)
            def _():
                unpack(g1 + 2)
                pltpu.async_copy(y_hbm.at[row_r.at[lax.rem(g1 + 2, 4)]],
                                 gbuf1, gs1)

            return carry

        lax.fori_loop(0, nch // 2, eb, 0)

        plsc.subcore_barrier()
        pltpu.sync_copy(acc_sh.at[pl.ds(s * SLICE, SLICE)],
                        out_hbm.at[c, pl.ds(s * SLICE, SLICE)])

    return agg_kernel


def _prep_body(pr_ref, x_ref, dis_ref, y_ref):
    deg = pr_ref[0] + pr_ref[1]
    dis = jnp.where(deg > 0.0, lax.rsqrt(deg), 0.0)
    dis_ref[...] = dis
    y_ref[...] = x_ref[...] * dis[:N_NODES]


def _final_body(q_ref, dis_ref, w1t_ref, b1_ref, wat_ref, ba_ref,
                wot_ref, bo_ref, out_ref, acc_ref, m_ref, z_ref):
    i = pl.program_id(0)

    @pl.when(i == 0)
    def _init():
        m_ref[0] = -jnp.inf
        z_ref[0] = 0.0
        acc_ref[...] = jnp.zeros_like(acc_ref)

    agg = (q_ref[0] + q_ref[1]) * dis_ref[...]
    h = jnp.dot(agg, w1t_ref[...], preferred_element_type=jnp.float32)
    h = jnp.maximum(h + b1_ref[...], 0.0)
    sv = jnp.dot(h, wat_ref[...], preferred_element_type=jnp.float32)
    sv = sv + ba_ref[...]
    bm = jnp.max(sv)
    m_old = m_ref[0]
    m_new = jnp.maximum(m_old, bm)
    alpha = jnp.exp(m_old - m_new)
    w = jnp.exp(sv - m_new)
    z_ref[0] = z_ref[0] * alpha + jnp.sum(w)
    acc_ref[...] = acc_ref[...] * alpha + jnp.sum(w * h, axis=0, keepdims=True)
    m_ref[0] = m_new

    @pl.when(i == pl.num_programs(0) - 1)
    def _fin():
        pooled = acc_ref[...] / z_ref[0]
        out_ref[...] = jnp.dot(pooled, wot_ref[...],
                               preferred_element_type=jnp.float32) + bo_ref[...]


def kernel(x, edge_index, W1, b1, Wa, ba, Wo, bo):
    n, d_in = x.shape
    d_hid = W1.shape[0]
    d_out = Wo.shape[0]
    e = edge_index.shape[1]

    ei = edge_index.astype(jnp.int32)
    ar = jnp.arange(n, dtype=jnp.int32)
    row = jnp.concatenate([ei[0], ar, ar])
    col = jnp.concatenate([ei[1], ar, ar])
    etot = e + 2 * n
    ch_sum = -(-etot // (16 * CK))
    ch0 = max(2, int(round(ch_sum * CORE0_FRAC / 2.0)) * 2)
    ch1 = max(2, (ch_sum - ch0 + 1) // 2 * 2)
    chmax = max(ch0, ch1)
    packed = row * 16384 + col
    npad = 16 * (ch0 + ch1) * CK - etot
    deadvals = DEAD + (jnp.arange(max(npad, chmax * CK), dtype=jnp.int32)
                       % (NPAD - DEAD))
    packed = jnp.concatenate([packed, deadvals[:npad]])
    parts = []
    off = 0
    for t in range(NT):
        cnt = (ch0 if t < 16 else ch1) * CK
        seg = packed[off:off + cnt]
        off += cnt
        if cnt < chmax * CK:
            seg = jnp.concatenate([seg, deadvals[:chmax * CK - cnt]])
        parts.append(seg)
    pk3 = jnp.stack(parts).reshape(NT, chmax, CK)

    _, deg_partial = _make_deg(chmax)(pk3)
    pr2 = deg_partial.reshape(2, NPAD, 1)
    dis_col, y = pl.pallas_call(
        _prep_body,
        out_shape=(jax.ShapeDtypeStruct((NPAD, 1), jnp.float32),
                   jax.ShapeDtypeStruct((n, d_in), jnp.float32)),
    )(pr2, x)
    q = _make_agg(ch0, ch1)(pk3, y)

    out = pl.pallas_call(
        _final_body,
        grid=(n // BN,),
        in_specs=[
            pl.BlockSpec((2, BN, d_in), lambda i: (0, i, 0)),
            pl.BlockSpec((BN, 1), lambda i: (i, 0)),
            pl.BlockSpec((d_in, d_hid), lambda i: (0, 0)),
            pl.BlockSpec((1, d_hid), lambda i: (0, 0)),
            pl.BlockSpec((d_hid, 1), lambda i: (0, 0)),
            pl.BlockSpec((1, 1), lambda i: (0, 0)),
            pl.BlockSpec((d_hid, d_out), lambda i: (0, 0)),
            pl.BlockSpec((1, d_out), lambda i: (0, 0)),
        ],
        out_specs=pl.BlockSpec((1, d_out), lambda i: (0, 0)),
        out_shape=jax.ShapeDtypeStruct((1, d_out), jnp.float32),
        scratch_shapes=[
            pltpu.VMEM((1, d_hid), jnp.float32),
            pltpu.SMEM((1,), jnp.float32),
            pltpu.SMEM((1,), jnp.float32),
        ],
    )(q, dis_col, W1.T, b1.reshape(1, d_hid), Wa.T, ba.reshape(1, 1),
      Wo.T, bo.reshape(1, d_out))
    return out


# speedup vs baseline: 26.4958x; 1.2132x over previous
import functools

import jax
import jax.numpy as jnp
from jax import lax
from jax.experimental import pallas as pl
from jax.experimental.pallas import tpu as pltpu
from jax.experimental.pallas import tpu_sc as plsc

N_NODES = 10000
NPAD = 10240
SLICE = NPAD // 16
DEAD = 10016
D_IN = 128
NT = 32
LANES = 16
CK = 96
CORE0_FRAC = 0.50
BN = 2000

def _mesh():
    return plsc.VectorSubcoreMesh(core_axis_name="c", subcore_axis_name="s")


@functools.lru_cache(maxsize=None)
def _make_deg(ch):
    @functools.partial(
        pl.kernel,
        mesh=_mesh(),
        compiler_params=pltpu.CompilerParams(
            needs_layout_passes=False, use_tc_tiling_on_sc=False),
        out_type=(jax.ShapeDtypeStruct((NT, NPAD), jnp.float32),
                  jax.ShapeDtypeStruct((2, NPAD), jnp.float32)),
        scratch_types=[
            pltpu.VMEM((ch, CK), jnp.int32),
            pltpu.VMEM((NPAD,), jnp.float32),
            pltpu.VMEM((SLICE,), jnp.float32),
            pltpu.VMEM((SLICE,), jnp.float32),
        ],
    )
    def deg_kernel(pk_hbm, stage_hbm, out_hbm, pk_v, deg_v, acc_v, tmp_v):
        c = lax.axis_index("c")
        s = lax.axis_index("s")
        wid = c * 16 + s
        pltpu.sync_copy(pk_hbm.at[wid], pk_v)
        zero = jnp.zeros((LANES,), jnp.float32)
        one = jnp.ones((LANES,), jnp.float32)

        def zb(i, carry):
            deg_v[pl.ds(i * LANES, LANES)] = zero
            return carry

        lax.fori_loop(0, NPAD // LANES, zb, 0)

        def hb(i, carry):
            p = pk_v[i // (CK // LANES), pl.ds((i % (CK // LANES)) * LANES, LANES)]
            idx = lax.bitwise_and(p, 16383)
            plsc.addupdate_scatter(deg_v, [idx], one)
            return carry

        lax.fori_loop(0, ch * (CK // LANES), hb, 0)

        pltpu.sync_copy(deg_v, stage_hbm.at[wid])
        plsc.subcore_barrier()

        def za(i, carry):
            acc_v[pl.ds(i * LANES, LANES)] = zero
            return carry

        lax.fori_loop(0, SLICE // LANES, za, 0)

        def rb(t, carry):
            pltpu.sync_copy(stage_hbm.at[c * 16 + t, pl.ds(s * SLICE, SLICE)],
                            tmp_v)

            def ab(i, c2):
                sl = pl.ds(i * LANES, LANES)
                acc_v[sl] = acc_v[sl] + tmp_v[sl]
                return c2

            return lax.fori_loop(0, SLICE // LANES, ab, carry)

        lax.fori_loop(0, 16, rb, 0)
        pltpu.sync_copy(acc_v, out_hbm.at[c, pl.ds(s * SLICE, SLICE)])

    return deg_kernel


@functools.lru_cache(maxsize=None)
def _make_agg(ch0, ch1):
    chmax = max(ch0, ch1)

    @functools.partial(
        pl.kernel,
        mesh=_mesh(),
        compiler_params=pltpu.CompilerParams(
            needs_layout_passes=False, use_tc_tiling_on_sc=False),
        out_type=jax.ShapeDtypeStruct((2, NPAD, D_IN), jnp.float32),
        scratch_types=[
            pltpu.VMEM((chmax, CK), jnp.int32),
            pltpu.VMEM((4, CK), jnp.int32),
            pltpu.VMEM((4, CK), jnp.int32),
            pltpu.VMEM((CK, D_IN), jnp.float32),
            pltpu.VMEM((CK, D_IN), jnp.float32),
            pltpu.VMEM_SHARED((NPAD, D_IN), jnp.float32),
            pltpu.SemaphoreType.DMA,
            pltpu.SemaphoreType.DMA,
            pltpu.SemaphoreType.DMA,
            pltpu.SemaphoreType.DMA,
        ],
    )
    def agg_kernel(pk_hbm, y_hbm, out_hbm, pk_v, col_r, row_r, gbuf0, gbuf1,
                   acc_sh, gs0, gs1, ss0, ss1):
        c = lax.axis_index("c")
        s = lax.axis_index("s")
        wid = c * 16 + s
        nch = jnp.where(c == 0, ch0, ch1)
        pltpu.sync_copy(pk_hbm.at[wid], pk_v)

        zero = jnp.zeros((LANES,), jnp.float32)
        nsub = CK // LANES

        def unpack(g):
            slot = lax.rem(g, 4)

            def uu(j, carry):
                sl = pl.ds(j * LANES, LANES)
                p = pk_v[g, sl]
                row_r[slot, sl] = lax.shift_right_logical(p, 14)
                col_r[slot, sl] = lax.bitwise_and(p, 16383)
                return carry

            lax.fori_loop(0, nsub, uu, 0)

        def zb(i, carry):
            gbuf0[i // (D_IN // LANES),
                  pl.ds((i % (D_IN // LANES)) * LANES, LANES)] = zero
            return carry

        lax.fori_loop(0, CK * (D_IN // LANES), zb, 0)
        for k in range(SLICE // CK):
            pltpu.sync_copy(gbuf0, acc_sh.at[pl.ds(s * SLICE + k * CK, CK)])
        rem = SLICE - (SLICE // CK) * CK
        if rem:
            pltpu.sync_copy(
                gbuf0.at[pl.ds(0, rem)],
                acc_sh.at[pl.ds(s * SLICE + (SLICE // CK) * CK, rem)])

        unpack(jnp.int32(0))
        unpack(jnp.int32(1))
        pltpu.async_copy(y_hbm.at[row_r.at[0]], gbuf0, gs0)
        pltpu.async_copy(y_hbm.at[row_r.at[1]], gbuf1, gs1)
        plsc.subcore_barrier()

        def eb(i, carry):
            g0 = i * 2
            g1 = g0 + 1
            s0 = lax.rem(g0, 4)
            s1 = lax.rem(g1, 4)
            pltpu.make_async_copy(y_hbm.at[row_r.at[s0]], gbuf0, gs0).wait()
            pltpu.async_copy(gbuf0, acc_sh.at[col_r.at[s0]], ss0, add=True)
            pltpu.make_async_copy(y_hbm.at[row_r.at[s1]], gbuf1, gs1).wait()
            pltpu.async_copy(gbuf1, acc_sh.at[col_r.at[s1]], ss1, add=True)
            pltpu.make_async_copy(gbuf0, acc_sh.at[col_r.at[s0]], ss0).wait()

            @pl.when(g0 + 2 < nch)
            def _():
                unpack(g0 + 2)
                pltpu.async_copy(y_hbm.at[row_r.at[lax.rem(g0 + 2, 4)]],
                                 gbuf0, gs0)

            pltpu.make_async_copy(gbuf1, acc_sh.at[col_r.at[s1]], ss1).wait()

            @pl.when(g1 + 2 < nch)
            def _():
                unpack(g1 + 2)
                pltpu.async_copy(y_hbm.at[row_r.at[lax.rem(g1 + 2, 4)]],
                                 gbuf1, gs1)

            return carry

        lax.fori_loop(0, nch // 2, eb, 0)

        plsc.subcore_barrier()
        pltpu.sync_copy(acc_sh.at[pl.ds(s * SLICE, SLICE)],
                        out_hbm.at[c, pl.ds(s * SLICE, SLICE)])

    return agg_kernel


def _prep_body(pr_ref, x_ref, dis_ref, y_ref):
    deg = pr_ref[0] + pr_ref[1]
    dis = jnp.where(deg > 0.0, lax.rsqrt(deg), 0.0)
    dis_ref[...] = dis
    y_ref[...] = x_ref[...] * dis[:N_NODES]


def _final_body(q_ref, dis_ref, w1t_ref, b1_ref, wat_ref, ba_ref,
                wot_ref, bo_ref, out_ref, acc_ref, m_ref, z_ref):
    i = pl.program_id(0)

    @pl.when(i == 0)
    def _init():
        m_ref[0] = -jnp.inf
        z_ref[0] = 0.0
        acc_ref[...] = jnp.zeros_like(acc_ref)

    agg = (q_ref[0] + q_ref[1]) * dis_ref[...]
    h = jnp.dot(agg, w1t_ref[...], preferred_element_type=jnp.float32)
    h = jnp.maximum(h + b1_ref[...], 0.0)
    sv = jnp.dot(h, wat_ref[...], preferred_element_type=jnp.float32)
    sv = sv + ba_ref[...]
    bm = jnp.max(sv)
    m_old = m_ref[0]
    m_new = jnp.maximum(m_old, bm)
    alpha = jnp.exp(m_old - m_new)
    w = jnp.exp(sv - m_new)
    z_ref[0] = z_ref[0] * alpha + jnp.sum(w)
    acc_ref[...] = acc_ref[...] * alpha + jnp.sum(w * h, axis=0, keepdims=True)
    m_ref[0] = m_new

    @pl.when(i == pl.num_programs(0) - 1)
    def _fin():
        pooled = acc_ref[...] / z_ref[0]
        out_ref[...] = jnp.dot(pooled, wot_ref[...],
                               preferred_element_type=jnp.float32) + bo_ref[...]


def kernel(x, edge_index, W1, b1, Wa, ba, Wo, bo):
    n, d_in = x.shape
    d_hid = W1.shape[0]
    d_out = Wo.shape[0]
    e = edge_index.shape[1]

    ei = edge_index.astype(jnp.int32)
    ar = jnp.arange(n, dtype=jnp.int32)
    row = jnp.concatenate([ei[0], ar, ar])
    col = jnp.concatenate([ei[1], ar, ar])
    etot = e + 2 * n
    ch_sum = -(-etot // (16 * CK))
    ch0 = max(2, int(round(ch_sum * CORE0_FRAC / 2.0)) * 2)
    ch1 = max(2, (ch_sum - ch0 + 1) // 2 * 2)
    chmax = max(ch0, ch1)
    packed = row * 16384 + col
    npad = 16 * (ch0 + ch1) * CK - etot
    deadvals = DEAD + (jnp.arange(max(npad, chmax * CK), dtype=jnp.int32)
                       % (NPAD - DEAD))
    packed = jnp.concatenate([packed, deadvals[:npad]])
    parts = []
    off = 0
    for t in range(NT):
        cnt = (ch0 if t < 16 else ch1) * CK
        seg = packed[off:off + cnt]
        off += cnt
        if cnt < chmax * CK:
            seg = jnp.concatenate([seg, deadvals[:chmax * CK - cnt]])
        parts.append(seg)
    pk3 = jnp.stack(parts).reshape(NT, chmax, CK)

    _, deg_partial = _make_deg(chmax)(pk3)
    pr2 = deg_partial.reshape(2, NPAD, 1)
    dis_col, y = pl.pallas_call(
        _prep_body,
        out_shape=(jax.ShapeDtypeStruct((NPAD, 1), jnp.float32),
                   jax.ShapeDtypeStruct((n, d_in), jnp.float32)),
    )(pr2, x)
    q = _make_agg(ch0, ch1)(pk3, y)

    out = pl.pallas_call(
        _final_body,
        grid=(n // BN,),
        in_specs=[
            pl.BlockSpec((2, BN, d_in), lambda i: (0, i, 0)),
            pl.BlockSpec((BN, 1), lambda i: (i, 0)),
            pl.BlockSpec((d_in, d_hid), lambda i: (0, 0)),
            pl.BlockSpec((1, d_hid), lambda i: (0, 0)),
            pl.BlockSpec((d_hid, 1), lambda i: (0, 0)),
            pl.BlockSpec((1, 1), lambda i: (0, 0)),
            pl.BlockSpec((d_hid, d_out), lambda i: (0, 0)),
            pl.BlockSpec((1, d_out), lambda i: (0, 0)),
        ],
        out_specs=pl.BlockSpec((1, d_out), lambda i: (0, 0)),
        out_shape=jax.ShapeDtypeStruct((1, d_out), jnp.float32),
        scratch_shapes=[
            pltpu.VMEM((1, d_hid), jnp.float32),
            pltpu.SMEM((1,), jnp.float32),
            pltpu.SMEM((1,), jnp.float32),
        ],
    )(q, dis_col, W1.T, b1.reshape(1, d_hid), Wa.T, ba.reshape(1, 1),
      Wo.T, bo.reshape(1, d_out))
    return out
